# baseline (speedup 1.0000x reference)
import functools

import jax
import jax.numpy as jnp
from jax import lax
from jax.experimental import pallas as pl
from jax.experimental.pallas import tpu as pltpu
from jax.experimental.pallas import tpu_sc as plsc

B = 128
N = 512
F = 64
PE = 500
GE = 5000
HID = 128
GED = 128
CLIN = 6
NCLS = 2

BP = 8
NWORK = 32
PPW = B // NWORK
KBLK = N // 128
PLANE = KBLK * N * 128
ZCH = 65536
PEP = 512
GEP = 5120


def _sc_build_body(pe_hbm, ge_hbm, outp_hbm, outg_hbm,
                   zbuf, ebuf, ibuf, vbuf, sem):
    wid = lax.axis_index("s") * 2 + lax.axis_index("c")

    def _fill(i, _):
        zbuf[pl.ds(i * 16, 16)] = jnp.zeros((16,), jnp.float32)
        vbuf[pl.ds(jnp.minimum(i, 31) * 16, 16)] = jnp.ones((16,), jnp.float32)
        return 0

    lax.fori_loop(0, ZCH // 16, _fill, 0, unroll=8)

    base = wid * (PPW * PLANE)

    def _zero(i, _):
        pltpu.sync_copy(zbuf, outp_hbm.at[pl.ds(base + i * ZCH, ZCH)])
        return 0

    lax.fori_loop(0, PPW * PLANE // ZCH, _zero, 0)

    @pl.when(wid == 0)
    def _():
        pltpu.sync_copy(zbuf.at[pl.ds(0, B * B)], outg_hbm)

    for j in range(PPW):
        b = wid * PPW + j
        pltpu.sync_copy(pe_hbm.at[b], ebuf)
        pbase = b * PLANE

        def _pidx(c, _):
            s = ebuf[0, pl.ds(c * 16, 16)]
            d = ebuf[1, pl.ds(c * 16, 16)]
            flat = (pbase + lax.shift_right_logical(s, 7) * (N * 128)
                    + d * 128 + lax.bitwise_and(s, 127))
            ibuf[pl.ds(c * 16, 16)] = flat
            return 0

        lax.fori_loop(0, PEP // 16, _pidx, 0, unroll=4)
        pltpu.async_copy(vbuf, outp_hbm.at[ibuf], sem).wait()

    @pl.when(wid == 0)
    def _():
        for g in range(GEP // PEP):
            pltpu.sync_copy(ge_hbm.at[0, pl.ds(g * PEP, PEP)], ebuf.at[0])
            pltpu.sync_copy(ge_hbm.at[1, pl.ds(g * PEP, PEP)], ebuf.at[1])

            def _gidx(c, _):
                s = ebuf[0, pl.ds(c * 16, 16)]
                d = ebuf[1, pl.ds(c * 16, 16)]
                ibuf[pl.ds(c * 16, 16)] = d * B + s
                return 0

            lax.fori_loop(0, PEP // 16, _gidx, 0, unroll=4)
            pltpu.async_copy(vbuf, outg_hbm.at[ibuf], sem).wait()


@functools.partial(
    pl.kernel,
    out_type=(jax.ShapeDtypeStruct((B * PLANE,), jnp.float32),
              jax.ShapeDtypeStruct((B * B,), jnp.float32)),
    mesh=plsc.VectorSubcoreMesh(core_axis_name="c", subcore_axis_name="s"),
    scratch_types=[
        pltpu.VMEM((ZCH,), jnp.float32),
        pltpu.VMEM((2, PEP), jnp.int32),
        pltpu.VMEM((PEP,), jnp.int32),
        pltpu.VMEM((PEP,), jnp.float32),
        pltpu.SemaphoreType.DMA,
    ],
)
def _sc_build(pe_hbm, ge_hbm, outp_hbm, outg_hbm, zbuf, ebuf, ibuf, vbuf, sem):
    _sc_build_body(pe_hbm, ge_hbm, outp_hbm, outg_hbm,
                   zbuf, ebuf, ibuf, vbuf, sem)


def _patient_body(a_ref, x_ref, w1_ref, b1_ref, w2_ref, b2_ref,
                  w3_ref, b3_ref, plw_ref, plb_ref, out_ref):
    w1 = w1_ref[...]
    w2 = w2_ref[...]
    w3 = w3_ref[...]
    b1 = b1_ref[...]
    b2 = b2_ref[...]
    b3 = b3_ref[...]
    for p in range(BP):
        deg = jnp.full((N, 1), 1.0, jnp.float32)
        for k in range(KBLK):
            deg += jnp.sum(a_ref[p, k], axis=1, keepdims=True)
        dis = lax.rsqrt(deg)
        dis2 = dis * dis
        h = x_ref[p]
        for w, bb in ((w1, b1), (w2, b2), (w3, b3)):
            z = jnp.dot(h, w, preferred_element_type=jnp.float32)
            zn = dis * z
            acc = dis2 * z + bb
            for k in range(KBLK):
                acc += jnp.dot(a_ref[p, k], zn[k * 128:(k + 1) * 128, :],
                               preferred_element_type=jnp.float32) * dis
            h = jnp.maximum(acc, 0.0)
        g = jnp.max(h, axis=0, keepdims=True)
        out_ref[p:p + 1, :] = (
            jnp.dot(g, plw_ref[...], preferred_element_type=jnp.float32)
            + plb_ref[...])


def _group_body(ag_ref, emb_ref, demo_ref, w1a_ref, w1b_ref, b1_ref,
                w2_ref, b2_ref, w3_ref, b3_ref, w4_ref, b4_ref, out_ref):
    abar = ag_ref[...]
    deg = jnp.sum(abar, axis=1, keepdims=True) + 1.0
    dis = lax.rsqrt(deg)
    dis2 = dis * dis
    an = dis * abar * jnp.transpose(dis)

    z = (jnp.dot(emb_ref[...], w1a_ref[...], preferred_element_type=jnp.float32)
         + jnp.dot(demo_ref[...], w1b_ref[...], preferred_element_type=jnp.float32))
    h = jnp.maximum(jnp.dot(an, z, preferred_element_type=jnp.float32)
                    + dis2 * z + b1_ref[...], 0.0)
    for w_ref, b_ref, act in ((w2_ref, b2_ref, True), (w3_ref, b3_ref, True),
                              (w4_ref, b4_ref, False)):
        z = jnp.dot(h, w_ref[...], preferred_element_type=jnp.float32)
        h = (jnp.dot(an, z, preferred_element_type=jnp.float32)
             + dis2 * z + b_ref[...])
        if act:
            h = jnp.maximum(h, 0.0)
    m = jnp.max(h, axis=1, keepdims=True)
    y = h - m
    out_ref[...] = y - jnp.log(jnp.sum(jnp.exp(y), axis=1, keepdims=True))


def kernel(x, demographic, patient_edge_idx, group_edge_idx,
           pW1, pb1, pW2, pb2, pW3, pb3, plinW, plinb,
           gW1, gb1, gW2, gb2, gW3, gb3, gW4, gb4):
    pe_pad = jnp.concatenate(
        [patient_edge_idx,
         jnp.tile(patient_edge_idx[:, :, -1:], (1, 1, PEP - PE))], axis=2)
    ge_pad = jnp.concatenate(
        [group_edge_idx,
         jnp.tile(group_edge_idx[:, -1:], (1, GEP - GE))], axis=1)

    abar_flat, ag_flat = _sc_build(pe_pad, ge_pad)
    abar = abar_flat.reshape(B, KBLK, N, 128)
    ag = ag_flat.reshape(B, B)

    row = lambda v: v.reshape(1, -1)
    fullg = lambda a: pl.BlockSpec(a.shape, lambda i: (0,) * a.ndim)
    full = lambda a: pl.BlockSpec(a.shape, lambda: (0,) * a.ndim)

    wspecs = [fullg(a) for a in (pW1, row(pb1), pW2, row(pb2), pW3, row(pb3),
                                 plinW, row(plinb))]
    embed = pl.pallas_call(
        _patient_body,
        grid=(B // BP,),
        in_specs=[
            pl.BlockSpec((BP, KBLK, N, 128), lambda i: (i, 0, 0, 0)),
            pl.BlockSpec((BP, N, F), lambda i: (i, 0, 0)),
        ] + wspecs,
        out_specs=pl.BlockSpec((BP, GED), lambda i: (i, 0)),
        out_shape=jax.ShapeDtypeStruct((B, GED), jnp.float32),
    )(abar, x, pW1, row(pb1), pW2, row(pb2), pW3, row(pb3),
      plinW, row(plinb))

    gw1a = gW1[:GED]
    gw1b = gW1[GED:]
    gargs = (ag, embed, demographic, gw1a, gw1b, row(gb1),
             gW2, row(gb2), gW3, row(gb3), gW4, row(gb4))
    out = pl.pallas_call(
        _group_body,
        in_specs=[full(a) for a in gargs],
        out_specs=pl.BlockSpec((B, NCLS), lambda: (0, 0)),
        out_shape=jax.ShapeDtypeStruct((B, NCLS), jnp.float32),
    )(*gargs)
    return out
